# baseline (speedup 1.0000x reference)
import jax
import jax.numpy as jnp
from jax.experimental import pallas as pl

_BN = 160


def _body(x_ref, wc_ref, bc_ref, wr_ref, br_ref, cls_ref, reg_ref):
    s = x_ref.shape[2]
    xm = jnp.sum(x_ref[...], axis=2) * (1.0 / s)
    cls_ref[...] = (
        jnp.dot(xm, wc_ref[...], preferred_element_type=jnp.float32) + bc_ref[...]
    )
    reg_ref[...] = (
        jnp.dot(xm, wr_ref[...], preferred_element_type=jnp.float32) + br_ref[...]
    )


def kernel(x, W_cls, b_cls, W_reg, b_reg):
    n, c, rh, rw = x.shape
    s = rh * rw
    k1 = W_cls.shape[0]
    k2 = W_reg.shape[0]
    x3 = x.reshape(n, c, s)
    wct = W_cls.T
    wrt = W_reg.T
    bc2 = b_cls.reshape(1, k1)
    br2 = b_reg.reshape(1, k2)
    cls, reg = pl.pallas_call(
        _body,
        grid=(n // _BN,),
        in_specs=[
            pl.BlockSpec((_BN, c, s), lambda i: (i, 0, 0)),
            pl.BlockSpec((c, k1), lambda i: (0, 0)),
            pl.BlockSpec((1, k1), lambda i: (0, 0)),
            pl.BlockSpec((c, k2), lambda i: (0, 0)),
            pl.BlockSpec((1, k2), lambda i: (0, 0)),
        ],
        out_specs=[
            pl.BlockSpec((_BN, k1), lambda i: (i, 0)),
            pl.BlockSpec((_BN, k2), lambda i: (i, 0)),
        ],
        out_shape=[
            jax.ShapeDtypeStruct((n, k1), jnp.float32),
            jax.ShapeDtypeStruct((n, k2), jnp.float32),
        ],
    )(x3, wct, bc2, wrt, br2)
    return (cls, reg)
